# baseline (speedup 1.0000x reference)
import jax
import jax.numpy as jnp
from jax.experimental import pallas as pl
from jax.experimental.pallas import tpu as pltpu

_BETA = 0.25
_NUM_LEVELS = 3
_K = 1024
_E = 64
_B_BLK = 512


def _fused_kernel(x_ref, W0_ref, b0_ref, W1_ref, b1_ref, W2_ref, b2_ref,
                  cb_ref, cn_ref, xq_ref, codes_ref, loss_ref):
    h = jnp.maximum(jnp.dot(x_ref[...], W0_ref[...],
                            preferred_element_type=jnp.float32) + b0_ref[...], 0.0)
    h = jnp.maximum(jnp.dot(h, W1_ref[...],
                            preferred_element_type=jnp.float32) + b1_ref[...], 0.0)
    z = jnp.dot(h, W2_ref[...], preferred_element_type=jnp.float32) + b2_ref[...]

    n = z.shape[0]
    lane_iota = jax.lax.broadcasted_iota(jnp.int32, (n, _K), 1)

    res = z
    xq = jnp.zeros_like(z)
    loss_sum = jnp.zeros((), dtype=jnp.float32)
    codes = []
    for i in range(_NUM_LEVELS):
        cb = cb_ref[i]
        cn = cn_ref[i]
        rnorm = jnp.sum(res * res, axis=1, keepdims=True)
        mm = jax.lax.dot_general(res, cb, (((1,), (1,)), ((), ())),
                                 preferred_element_type=jnp.float32)
        d = (rnorm - 2.0 * mm) + cn
        dmin = jnp.min(d, axis=1, keepdims=True)
        idx = jnp.min(jnp.where(d == dmin, lane_iota, _K),
                      axis=1, keepdims=True)
        onehot = (lane_iota == idx).astype(jnp.float32)
        q = jnp.dot(onehot, cb, preferred_element_type=jnp.float32,
                    precision=jax.lax.Precision.HIGHEST)
        diff = q - res
        loss_sum = loss_sum + jnp.sum(diff * diff)
        q_st = res + (q - res)
        res = res - q_st
        xq = xq + q_st
        codes.append(idx)

    xq_ref[...] = xq
    codes_ref[...] = jnp.concatenate(codes, axis=1)

    @pl.when(pl.program_id(0) == 0)
    def _init():
        loss_ref[...] = jnp.zeros_like(loss_ref)

    loss_ref[...] += jnp.reshape(loss_sum, (1, 1))


def kernel(x, W0, b0, W1, b1, W2, b2, codebooks):
    B = x.shape[0]
    cn = jnp.sum(codebooks ** 2, axis=-1)[:, None, :]
    grid = (B // _B_BLK,)
    xq, codes, loss = pl.pallas_call(
        _fused_kernel,
        grid=grid,
        in_specs=[
            pl.BlockSpec((_B_BLK, 768), lambda i: (i, 0)),
            pl.BlockSpec((768, 512), lambda i: (0, 0)),
            pl.BlockSpec((1, 512), lambda i: (0, 0)),
            pl.BlockSpec((512, 256), lambda i: (0, 0)),
            pl.BlockSpec((1, 256), lambda i: (0, 0)),
            pl.BlockSpec((256, _E), lambda i: (0, 0)),
            pl.BlockSpec((1, _E), lambda i: (0, 0)),
            pl.BlockSpec((_NUM_LEVELS, _K, _E), lambda i: (0, 0, 0)),
            pl.BlockSpec((_NUM_LEVELS, 1, _K), lambda i: (0, 0, 0)),
        ],
        out_specs=[
            pl.BlockSpec((_B_BLK, _E), lambda i: (i, 0)),
            pl.BlockSpec((_B_BLK, _NUM_LEVELS), lambda i: (i, 0)),
            pl.BlockSpec((1, 1), lambda i: (0, 0)),
        ],
        out_shape=[
            jax.ShapeDtypeStruct((B, _E), jnp.float32),
            jax.ShapeDtypeStruct((B, _NUM_LEVELS), jnp.int32),
            jax.ShapeDtypeStruct((1, 1), jnp.float32),
        ],
        compiler_params=pltpu.CompilerParams(
            dimension_semantics=("arbitrary",),
        ),
    )(x, W0, b0[None, :], W1, b1[None, :], W2, b2[None, :], codebooks, cn)

    rq_loss = loss[0, 0] * ((1.0 + _BETA) / (_NUM_LEVELS * B * _E))
    return xq, rq_loss, codes


# speedup vs baseline: 1.4182x; 1.4182x over previous
import jax
import jax.numpy as jnp
from jax.experimental import pallas as pl
from jax.experimental.pallas import tpu as pltpu

_BETA = 0.25
_NUM_LEVELS = 3
_K = 1024
_E = 64
_B_BLK = 512


def _fused_kernel(x_ref, W0_ref, b0_ref, W1_ref, b1_ref, W2_ref, b2_ref,
                  cb_ref, cbh_ref, cbm_ref, cbl_ref, cn_ref, xq_ref,
                  codes_ref, loss_ref):
    h = jnp.maximum(jnp.dot(x_ref[...], W0_ref[...],
                            preferred_element_type=jnp.float32) + b0_ref[...], 0.0)
    h = jnp.maximum(jnp.dot(h, W1_ref[...],
                            preferred_element_type=jnp.float32) + b1_ref[...], 0.0)
    z = jnp.dot(h, W2_ref[...], preferred_element_type=jnp.float32) + b2_ref[...]

    n = z.shape[0]
    lane_iota = jax.lax.broadcasted_iota(jnp.int32, (n, _K), 1)

    res = z
    xq = jnp.zeros_like(z)
    loss_sum = jnp.zeros((), dtype=jnp.float32)
    codes = []
    for i in range(_NUM_LEVELS):
        cb = cb_ref[i]
        cn = cn_ref[i]
        rnorm = jnp.sum(res * res, axis=1, keepdims=True)
        mm = jax.lax.dot_general(res, cb, (((1,), (1,)), ((), ())),
                                 preferred_element_type=jnp.float32)
        d = (rnorm - 2.0 * mm) + cn
        dmin = jnp.min(d, axis=1, keepdims=True)
        idx = jnp.min(jnp.where(d == dmin, lane_iota, _K),
                      axis=1, keepdims=True)
        onehot = (lane_iota == idx).astype(jnp.float32)
        q = ((jnp.dot(onehot, cbh_ref[i], preferred_element_type=jnp.float32)
              + jnp.dot(onehot, cbm_ref[i], preferred_element_type=jnp.float32))
             + jnp.dot(onehot, cbl_ref[i], preferred_element_type=jnp.float32))
        diff = q - res
        loss_sum = loss_sum + jnp.sum(diff * diff)
        q_st = res + (q - res)
        res = res - q_st
        xq = xq + q_st
        codes.append(idx)

    xq_ref[...] = xq
    codes_ref[...] = jnp.concatenate(codes, axis=1)

    @pl.when(pl.program_id(0) == 0)
    def _init():
        loss_ref[...] = jnp.zeros_like(loss_ref)

    loss_ref[...] += jnp.reshape(loss_sum, (1, 1))


def kernel(x, W0, b0, W1, b1, W2, b2, codebooks):
    B = x.shape[0]
    cn = jnp.sum(codebooks ** 2, axis=-1)[:, None, :]
    cbh = codebooks.astype(jnp.bfloat16).astype(jnp.float32)
    cbm = (codebooks - cbh).astype(jnp.bfloat16).astype(jnp.float32)
    cbl = codebooks - cbh - cbm
    grid = (B // _B_BLK,)
    xq, codes, loss = pl.pallas_call(
        _fused_kernel,
        grid=grid,
        in_specs=[
            pl.BlockSpec((_B_BLK, 768), lambda i: (i, 0)),
            pl.BlockSpec((768, 512), lambda i: (0, 0)),
            pl.BlockSpec((1, 512), lambda i: (0, 0)),
            pl.BlockSpec((512, 256), lambda i: (0, 0)),
            pl.BlockSpec((1, 256), lambda i: (0, 0)),
            pl.BlockSpec((256, _E), lambda i: (0, 0)),
            pl.BlockSpec((1, _E), lambda i: (0, 0)),
            pl.BlockSpec((_NUM_LEVELS, _K, _E), lambda i: (0, 0, 0)),
            pl.BlockSpec((_NUM_LEVELS, _K, _E), lambda i: (0, 0, 0)),
            pl.BlockSpec((_NUM_LEVELS, _K, _E), lambda i: (0, 0, 0)),
            pl.BlockSpec((_NUM_LEVELS, _K, _E), lambda i: (0, 0, 0)),
            pl.BlockSpec((_NUM_LEVELS, 1, _K), lambda i: (0, 0, 0)),
        ],
        out_specs=[
            pl.BlockSpec((_B_BLK, _E), lambda i: (i, 0)),
            pl.BlockSpec((_B_BLK, _NUM_LEVELS), lambda i: (i, 0)),
            pl.BlockSpec((1, 1), lambda i: (0, 0)),
        ],
        out_shape=[
            jax.ShapeDtypeStruct((B, _E), jnp.float32),
            jax.ShapeDtypeStruct((B, _NUM_LEVELS), jnp.int32),
            jax.ShapeDtypeStruct((1, 1), jnp.float32),
        ],
        compiler_params=pltpu.CompilerParams(
            dimension_semantics=("arbitrary",),
        ),
    )(x, W0, b0[None, :], W1, b1[None, :], W2, b2[None, :], codebooks,
      cbh, cbm, cbl, cn)

    rq_loss = loss[0, 0] * ((1.0 + _BETA) / (_NUM_LEVELS * B * _E))
    return xq, rq_loss, codes


# speedup vs baseline: 1.5258x; 1.0758x over previous
import jax
import jax.numpy as jnp
from jax.experimental import pallas as pl
from jax.experimental.pallas import tpu as pltpu

_BETA = 0.25
_NUM_LEVELS = 3
_K = 1024
_E = 64
_B_BLK = 512


def _fused_kernel(x_ref, W0_ref, b0_ref, W1_ref, b1_ref, W2_ref, b2_ref,
                  cb_ref, cbh_ref, cbm_ref, cbl_ref, cn_ref, xq_ref,
                  codes_ref, loss_ref):
    h = jnp.maximum(jnp.dot(x_ref[...], W0_ref[...],
                            preferred_element_type=jnp.float32) + b0_ref[...], 0.0)
    h = jnp.maximum(jnp.dot(h, W1_ref[...],
                            preferred_element_type=jnp.float32) + b1_ref[...], 0.0)
    z = jnp.dot(h, W2_ref[...], preferred_element_type=jnp.float32) + b2_ref[...]

    n = z.shape[0]
    lane_iota = jax.lax.broadcasted_iota(jnp.int32, (n, _K), 1)

    res = z
    xq = jnp.zeros_like(z)
    loss_sum = jnp.zeros((), dtype=jnp.float32)
    codes = []
    for i in range(_NUM_LEVELS):
        cb = cb_ref[i]
        cn = cn_ref[i]
        rnorm = jnp.sum(res * res, axis=1, keepdims=True)
        mm = jax.lax.dot_general(res, cb, (((1,), (1,)), ((), ())),
                                 preferred_element_type=jnp.float32)
        d = (rnorm - 2.0 * mm) + cn
        dmin = jnp.min(d, axis=1, keepdims=True)
        idx = jnp.min(jnp.where(d == dmin, lane_iota, _K),
                      axis=1, keepdims=True)
        onehot = (lane_iota == idx).astype(jnp.float32)
        q = (jnp.dot(onehot, cbh_ref[i], preferred_element_type=jnp.float32)
             + jnp.dot(onehot, cbm_ref[i], preferred_element_type=jnp.float32))
        if i < _NUM_LEVELS - 1:
            q = q + jnp.dot(onehot, cbl_ref[i],
                            preferred_element_type=jnp.float32)
        diff = q - res
        loss_sum = loss_sum + jnp.sum(diff * diff)
        q_st = res + (q - res)
        res = res - q_st
        xq = xq + q_st
        codes.append(idx)

    xq_ref[...] = xq
    codes_ref[...] = jnp.concatenate(codes, axis=1)

    @pl.when(pl.program_id(0) == 0)
    def _init():
        loss_ref[...] = jnp.zeros_like(loss_ref)

    loss_ref[...] += jnp.reshape(loss_sum, (1, 1))


def kernel(x, W0, b0, W1, b1, W2, b2, codebooks):
    B = x.shape[0]
    cn = jnp.sum(codebooks ** 2, axis=-1)[:, None, :]
    cbh = codebooks.astype(jnp.bfloat16).astype(jnp.float32)
    cbm = (codebooks - cbh).astype(jnp.bfloat16).astype(jnp.float32)
    cbl = codebooks - cbh - cbm
    grid = (B // _B_BLK,)
    xq, codes, loss = pl.pallas_call(
        _fused_kernel,
        grid=grid,
        in_specs=[
            pl.BlockSpec((_B_BLK, 768), lambda i: (i, 0)),
            pl.BlockSpec((768, 512), lambda i: (0, 0)),
            pl.BlockSpec((1, 512), lambda i: (0, 0)),
            pl.BlockSpec((512, 256), lambda i: (0, 0)),
            pl.BlockSpec((1, 256), lambda i: (0, 0)),
            pl.BlockSpec((256, _E), lambda i: (0, 0)),
            pl.BlockSpec((1, _E), lambda i: (0, 0)),
            pl.BlockSpec((_NUM_LEVELS, _K, _E), lambda i: (0, 0, 0)),
            pl.BlockSpec((_NUM_LEVELS, _K, _E), lambda i: (0, 0, 0)),
            pl.BlockSpec((_NUM_LEVELS, _K, _E), lambda i: (0, 0, 0)),
            pl.BlockSpec((_NUM_LEVELS, _K, _E), lambda i: (0, 0, 0)),
            pl.BlockSpec((_NUM_LEVELS, 1, _K), lambda i: (0, 0, 0)),
        ],
        out_specs=[
            pl.BlockSpec((_B_BLK, _E), lambda i: (i, 0)),
            pl.BlockSpec((_B_BLK, _NUM_LEVELS), lambda i: (i, 0)),
            pl.BlockSpec((1, 1), lambda i: (0, 0)),
        ],
        out_shape=[
            jax.ShapeDtypeStruct((B, _E), jnp.float32),
            jax.ShapeDtypeStruct((B, _NUM_LEVELS), jnp.int32),
            jax.ShapeDtypeStruct((1, 1), jnp.float32),
        ],
        compiler_params=pltpu.CompilerParams(
            dimension_semantics=("arbitrary",),
        ),
    )(x, W0, b0[None, :], W1, b1[None, :], W2, b2[None, :], codebooks,
      cbh, cbm, cbl, cn)

    rq_loss = loss[0, 0] * ((1.0 + _BETA) / (_NUM_LEVELS * B * _E))
    return xq, rq_loss, codes


# speedup vs baseline: 1.5341x; 1.0054x over previous
import jax
import jax.numpy as jnp
from jax.experimental import pallas as pl
from jax.experimental.pallas import tpu as pltpu

_BETA = 0.25
_NUM_LEVELS = 3
_K = 1024
_E = 64
_B_BLK = 512


def _fused_kernel(x_ref, W0_ref, b0_ref, W1_ref, b1_ref, W2_ref, b2_ref,
                  cb_ref, cbp_ref, cn_ref, xq_ref, codes_ref, loss_ref):
    h = jnp.maximum(jnp.dot(x_ref[...], W0_ref[...],
                            preferred_element_type=jnp.float32) + b0_ref[...], 0.0)
    h = jnp.maximum(jnp.dot(h, W1_ref[...],
                            preferred_element_type=jnp.float32) + b1_ref[...], 0.0)
    z = jnp.dot(h, W2_ref[...], preferred_element_type=jnp.float32) + b2_ref[...]

    n = z.shape[0]
    lane_iota = jax.lax.broadcasted_iota(jnp.int32, (n, _K), 1)

    res = z
    xq = jnp.zeros_like(z)
    loss_sum = jnp.zeros((), dtype=jnp.float32)
    codes = []
    for i in range(_NUM_LEVELS):
        cb = cb_ref[i]
        cn = cn_ref[i]
        rnorm = jnp.sum(res * res, axis=1, keepdims=True)
        mm = jax.lax.dot_general(res, cb, (((1,), (1,)), ((), ())),
                                 preferred_element_type=jnp.float32)
        d = (rnorm - 2.0 * mm) + cn
        dmin = jnp.min(d, axis=1, keepdims=True)
        idx = jnp.min(jnp.where(d == dmin, lane_iota, _K),
                      axis=1, keepdims=True)
        onehot = (lane_iota == idx).astype(jnp.float32)
        q = (jnp.dot(onehot, cbp_ref[i][:, :_E],
                     preferred_element_type=jnp.float32)
             + jnp.dot(onehot, cbp_ref[i][:, _E:2 * _E],
                       preferred_element_type=jnp.float32))
        if i < _NUM_LEVELS - 1:
            q = q + jnp.dot(onehot, cbp_ref[i][:, 2 * _E:],
                            preferred_element_type=jnp.float32)
        diff = q - res
        loss_sum = loss_sum + jnp.sum(diff * diff)
        q_st = res + (q - res)
        res = res - q_st
        xq = xq + q_st
        codes.append(idx)

    xq_ref[...] = xq
    codes_ref[...] = jnp.concatenate(codes, axis=1)

    @pl.when(pl.program_id(0) == 0)
    def _init():
        loss_ref[...] = jnp.zeros_like(loss_ref)

    loss_ref[...] += jnp.reshape(loss_sum, (1, 1))


def kernel(x, W0, b0, W1, b1, W2, b2, codebooks):
    B = x.shape[0]
    cn = jnp.sum(codebooks ** 2, axis=-1)[:, None, :]
    mask = jnp.int32(-65536)
    def _trunc_bf16(v):
        return jax.lax.bitcast_convert_type(
            jax.lax.bitcast_convert_type(v, jnp.int32) & mask, jnp.float32)
    cbh = _trunc_bf16(codebooks)
    r = codebooks - cbh
    cbm = _trunc_bf16(r)
    cbl = r - cbm
    cbp = jnp.concatenate([cbh, cbm, cbl], axis=-1)
    grid = (B // _B_BLK,)
    xq, codes, loss = pl.pallas_call(
        _fused_kernel,
        grid=grid,
        in_specs=[
            pl.BlockSpec((_B_BLK, 768), lambda i: (i, 0)),
            pl.BlockSpec((768, 512), lambda i: (0, 0)),
            pl.BlockSpec((1, 512), lambda i: (0, 0)),
            pl.BlockSpec((512, 256), lambda i: (0, 0)),
            pl.BlockSpec((1, 256), lambda i: (0, 0)),
            pl.BlockSpec((256, _E), lambda i: (0, 0)),
            pl.BlockSpec((1, _E), lambda i: (0, 0)),
            pl.BlockSpec((_NUM_LEVELS, _K, _E), lambda i: (0, 0, 0)),
            pl.BlockSpec((_NUM_LEVELS, _K, 3 * _E), lambda i: (0, 0, 0)),
            pl.BlockSpec((_NUM_LEVELS, 1, _K), lambda i: (0, 0, 0)),
        ],
        out_specs=[
            pl.BlockSpec((_B_BLK, _E), lambda i: (i, 0)),
            pl.BlockSpec((_B_BLK, _NUM_LEVELS), lambda i: (i, 0)),
            pl.BlockSpec((1, 1), lambda i: (0, 0)),
        ],
        out_shape=[
            jax.ShapeDtypeStruct((B, _E), jnp.float32),
            jax.ShapeDtypeStruct((B, _NUM_LEVELS), jnp.int32),
            jax.ShapeDtypeStruct((1, 1), jnp.float32),
        ],
        compiler_params=pltpu.CompilerParams(
            dimension_semantics=("arbitrary",),
        ),
    )(x, W0, b0[None, :], W1, b1[None, :], W2, b2[None, :], codebooks,
      cbp, cn)

    rq_loss = loss[0, 0] * ((1.0 + _BETA) / (_NUM_LEVELS * B * _E))
    return xq, rq_loss, codes
